# baseline (speedup 1.0000x reference)
import functools

import jax
import jax.numpy as jnp
from jax import lax
from jax.experimental import pallas as pl
from jax.experimental.pallas import tpu as pltpu
from jax.experimental.pallas import tpu_sc as plsc

N_NODES = 10000
D_FEAT = 256
D_LATENT = 128
N_EDGES = 160000

_NC = 2
_NS = 16
_NW = _NC * _NS
_L = 16

_EPW = N_EDGES // _NW
_CHUNK = 128
_NCH = -(-_EPW // _CHUNK)
_LAST = _EPW - _CHUNK
_NPAIR = _NCH // 2


def _encode_matmul(x, W):
    M, K = x.shape
    _, N = W.shape
    BM = 1024

    def body(x_ref, w_ref, z_ref, n_ref):
        z = jnp.dot(x_ref[...], w_ref[...], preferred_element_type=jnp.float32)
        z_ref[...] = z
        n_ref[...] = jnp.sum(z * z, axis=1)

    return pl.pallas_call(
        body,
        grid=(pl.cdiv(M, BM),),
        in_specs=[
            pl.BlockSpec((BM, K), lambda i: (i, 0)),
            pl.BlockSpec((K, N), lambda i: (0, 0)),
        ],
        out_specs=[
            pl.BlockSpec((BM, N), lambda i: (i, 0)),
            pl.BlockSpec((BM,), lambda i: (i,)),
        ],
        out_shape=[
            jax.ShapeDtypeStruct((M, N), jnp.float32),
            jax.ShapeDtypeStruct((M,), jnp.float32),
        ],
    )(x, W)


def _chunk_base(c):
    return jnp.minimum(c * _CHUNK, _LAST)


def _decode_body(z_hbm, n_hbm, src_hbm, dst_hbm, out_hbm,
                 idx_s, idx_d, buf0, buf1, ns0, nd0, ns1, nd1,
                 tbuf, out_v,
                 sem_r0, sem_a0, sem_n0, sem_r1, sem_a1, sem_n1):
    wid = lax.axis_index("s") * _NC + lax.axis_index("c")
    ebase = wid * _EPW

    pltpu.sync_copy(src_hbm.at[pl.ds(ebase, _EPW)], idx_s)
    pltpu.sync_copy(dst_hbm.at[pl.ds(ebase, _EPW)], idx_d)

    def s_slice(c):
        return idx_s.at[pl.ds(_chunk_base(c), _CHUNK)]

    def d_slice(c):
        return idx_d.at[pl.ds(_chunk_base(c), _CHUNK)]

    def issue_src(c, buf, sem):
        pltpu.async_copy(z_hbm.at[s_slice(c)], buf, sem)

    def wait_src(buf, sem):
        pltpu.make_async_copy(z_hbm.at[s_slice(0)], buf, sem).wait()

    def issue_add(c, buf, sem_a, ns, nd, sem_n):
        pltpu.async_copy(z_hbm.at[d_slice(c)], buf, sem_a, add=True)
        pltpu.async_copy(n_hbm.at[s_slice(c)], ns, sem_n)
        pltpu.async_copy(n_hbm.at[d_slice(c)], nd, sem_n)

    def wait_add(buf, sem_a, ns, nd, sem_n):
        pltpu.make_async_copy(z_hbm.at[d_slice(0)], buf, sem_a).wait()
        pltpu.make_async_copy(n_hbm.at[s_slice(0)], ns, sem_n).wait()
        pltpu.make_async_copy(n_hbm.at[d_slice(0)], nd, sem_n).wait()

    def compute(c, buf, ns, nd):
        b = _chunk_base(c)

        def group_body(g, carry):
            for e in range(_L):
                row = g * _L + e
                w0 = buf[row, pl.ds(0, _L)]
                w4 = buf[row, pl.ds(4 * _L, _L)]
                accs = [w0 * w0, w4 * w4]
                for k in range(D_LATENT // _L):
                    if k % 4 != 0:
                        kk = (k % 4) // 2
                        wk = buf[row, pl.ds(k * _L, _L)]
                        accs[kk] = accs[kk] + wk * wk
                tbuf[pl.ds(e * 17, _L)] = accs[0] + accs[1]
            rowv = lax.iota(jnp.int32, _L) * 17
            accs = [
                plsc.load_gather(tbuf, [rowv]),
                plsc.load_gather(tbuf, [rowv + 1]),
            ]
            for k in range(2, _L):
                accs[k % 2] = accs[k % 2] + plsc.load_gather(tbuf, [rowv + k])
            nsv = ns[pl.ds(g * _L, _L)]
            ndv = nd[pl.ds(g * _L, _L)]
            acc = 0.5 * (accs[0] + accs[1] - nsv - ndv)
            ex = jnp.exp(-jnp.abs(acc))
            sig = jnp.where(acc >= 0.0, 1.0 / (1.0 + ex), ex / (1.0 + ex))
            out_v[pl.ds(b + g * _L, _L)] = sig
            return carry

        lax.fori_loop(0, _CHUNK // _L, group_body, 0)

    issue_src(0, buf0, sem_r0)
    wait_src(buf0, sem_r0)
    issue_add(0, buf0, sem_a0, ns0, nd0, sem_n0)
    issue_src(1, buf1, sem_r1)

    def pair_body(p, carry):
        c0 = 2 * p

        def step(c, buf, sem_r, sem_a, ns, nd, sem_n,
                 nbuf, nsem_r, nsem_a, nns, nnd, nsem_n):
            @pl.when(c + 1 < _NCH)
            def _():
                wait_src(nbuf, nsem_r)
                issue_add(c + 1, nbuf, nsem_a, nns, nnd, nsem_n)

            wait_add(buf, sem_a, ns, nd, sem_n)
            compute(c, buf, ns, nd)

            @pl.when(c + 2 < _NCH)
            def _():
                issue_src(c + 2, buf, sem_r)

        step(c0, buf0, sem_r0, sem_a0, ns0, nd0, sem_n0,
             buf1, sem_r1, sem_a1, ns1, nd1, sem_n1)
        step(c0 + 1, buf1, sem_r1, sem_a1, ns1, nd1, sem_n1,
             buf0, sem_r0, sem_a0, ns0, nd0, sem_n0)
        return carry

    lax.fori_loop(0, _NPAIR, pair_body, 0)
    pltpu.sync_copy(out_v, out_hbm.at[pl.ds(ebase, _EPW)])


def _decode(z, n, src, dst):
    mesh = plsc.VectorSubcoreMesh(core_axis_name="c", subcore_axis_name="s")
    k = functools.partial(
        pl.kernel,
        out_type=jax.ShapeDtypeStruct((N_EDGES,), jnp.float32),
        mesh=mesh,
        scratch_types=[
            pltpu.VMEM((_EPW,), jnp.int32),
            pltpu.VMEM((_EPW,), jnp.int32),
            pltpu.VMEM((_CHUNK, D_LATENT), jnp.float32),
            pltpu.VMEM((_CHUNK, D_LATENT), jnp.float32),
            pltpu.VMEM((_CHUNK,), jnp.float32),
            pltpu.VMEM((_CHUNK,), jnp.float32),
            pltpu.VMEM((_CHUNK,), jnp.float32),
            pltpu.VMEM((_CHUNK,), jnp.float32),
            pltpu.VMEM((_L * 17,), jnp.float32),
            pltpu.VMEM((_EPW,), jnp.float32),
            pltpu.SemaphoreType.DMA,
            pltpu.SemaphoreType.DMA,
            pltpu.SemaphoreType.DMA,
            pltpu.SemaphoreType.DMA,
            pltpu.SemaphoreType.DMA,
            pltpu.SemaphoreType.DMA,
        ],
        compiler_params=pltpu.CompilerParams(needs_layout_passes=False),
    )(_decode_body)
    return k(z, n, src, dst)


def kernel(x, edge_index, W):
    z, n = _encode_matmul(x, W)
    ei = edge_index.astype(jnp.int32)
    return _decode(z, n, ei[0], ei[1])
